# Initial kernel scaffold; baseline (speedup 1.0000x reference)
import functools

import jax
import jax.numpy as jnp
from jax import lax
from jax.experimental import pallas as pl
from jax.experimental.pallas import tpu as pltpu
from jax.experimental.pallas import tpu_sc as plsc

N = 10000
E = 320000
IN_FEAT = 128
OUT_FEAT = 128
NUM_RELS = 90
NUM_BASES = 4
SUBMAT = 32

NC = 2
NS = 16
NW = NC * NS

C = 128
NCH = 80
EPW = C * NCH
E_PAD = NW * EPW
N_ACC = N + 16
ZROWS = 1000

TN = 2000
NT = N // TN


def _y_body(w_ref, x_ref, y_ref):
    w = w_ref[...]
    wcat = jnp.concatenate([w, w, w, w], axis=1)
    ri = lax.broadcasted_iota(jnp.int32, (128, 128), 0)
    ci = lax.broadcasted_iota(jnp.int32, (128, 128), 1)
    mask = (ri // SUBMAT) == (ci // SUBMAT)
    wbd = jnp.where(mask, wcat, 0.0)
    y_ref[...] = jnp.dot(x_ref[...], wbd, preferred_element_type=jnp.float32)


def _make_y(x, wr):
    return pl.pallas_call(
        _y_body,
        grid=(NT, NUM_RELS),
        in_specs=[
            pl.BlockSpec((128, 32), lambda nt, r: (r, 0)),
            pl.BlockSpec((TN, 128), lambda nt, r: (nt, 0)),
        ],
        out_specs=pl.BlockSpec((TN, 128), lambda nt, r: (r * NT + nt, 0)),
        out_shape=jax.ShapeDtypeStruct((NUM_RELS * N, 128), jnp.float32),
    )(wr, x)


def _sc_body(y_hbm, src_hbm, dst_hbm, typ_hbm, z_hbm, out_hbm,
             src_v, typ_v, dst_v, idx_v, rows_a, rows_b, h_sh, sem_a, sem_b):
    cid = lax.axis_index("c")
    sid = lax.axis_index("s")
    wid = cid * NS + sid

    @pl.when(sid < 10)
    def _zero():
        pltpu.sync_copy(z_hbm, h_sh.at[pl.ds(sid * ZROWS, ZROWS)])
    plsc.subcore_barrier()

    pltpu.sync_copy(src_hbm.at[wid], src_v)
    pltpu.sync_copy(typ_hbm.at[wid], typ_v)
    pltpu.sync_copy(dst_hbm.at[wid], dst_v)

    def _idx_row(j, carry):
        for k in range(C // 16):
            s = pl.ds(k * 16, 16)
            idx_v.at[j][s] = typ_v.at[j][s] * N + src_v.at[j][s]
        return carry
    lax.fori_loop(0, NCH, _idx_row, 0)

    def _gather(j, rows, sem):
        return pltpu.async_copy(y_hbm.at[idx_v.at[j]], rows, sem)

    def _wait(rows, sem):
        pltpu.make_async_copy(y_hbm.at[idx_v.at[0]], rows, sem).wait()

    def _scat(j, rows):
        pltpu.sync_copy(rows, h_sh.at[dst_v.at[j]], add=True)

    _gather(0, rows_a, sem_a)
    def _loop(j2, carry):
        j0 = j2 * 2
        _gather(j0 + 1, rows_b, sem_b)
        _wait(rows_a, sem_a)
        _scat(j0, rows_a)
        @pl.when(j2 < NCH // 2 - 1)
        def _():
            _gather(j0 + 2, rows_a, sem_a)
        _wait(rows_b, sem_b)
        _scat(j0 + 1, rows_b)
        return carry
    lax.fori_loop(0, NCH // 2, _loop, 0)

    plsc.subcore_barrier()

    @pl.when(sid < 10)
    def _writeout():
        pltpu.sync_copy(h_sh.at[pl.ds(sid * ZROWS, ZROWS)],
                        out_hbm.at[cid].at[pl.ds(sid * ZROWS, ZROWS)])


def _make_sc(y, src_r, dst_r, typ_r, zblk):
    mesh = plsc.VectorSubcoreMesh(core_axis_name="c", subcore_axis_name="s")
    f = pl.kernel(
        _sc_body,
        out_type=jax.ShapeDtypeStruct((NC, N, 128), jnp.float32),
        mesh=mesh,
        scratch_types=[
            pltpu.VMEM((NCH, C), jnp.int32),
            pltpu.VMEM((NCH, C), jnp.int32),
            pltpu.VMEM((NCH, C), jnp.int32),
            pltpu.VMEM((NCH, C), jnp.int32),
            pltpu.VMEM((C, 128), jnp.float32),
            pltpu.VMEM((C, 128), jnp.float32),
            pltpu.VMEM_SHARED((N_ACC, 128), jnp.float32),
            pltpu.SemaphoreType.DMA,
            pltpu.SemaphoreType.DMA,
        ],
    )
    return f(y, src_r, dst_r, typ_r, zblk)


def _fin_body(h_ref, x_ref, norm_ref, lw_ref, b_ref, o_ref):
    h = h_ref[0] + h_ref[1]
    lm = jnp.dot(x_ref[...], lw_ref[...], preferred_element_type=jnp.float32)
    o_ref[...] = h * norm_ref[...] + b_ref[...] + lm


def _make_fin(hpart, x, norm, loop_weight, bias2):
    return pl.pallas_call(
        _fin_body,
        grid=(NT,),
        in_specs=[
            pl.BlockSpec((NC, TN, 128), lambda i: (0, i, 0)),
            pl.BlockSpec((TN, 128), lambda i: (i, 0)),
            pl.BlockSpec((TN, 1), lambda i: (i, 0)),
            pl.BlockSpec((128, 128), lambda i: (0, 0)),
            pl.BlockSpec((1, 128), lambda i: (0, 0)),
        ],
        out_specs=pl.BlockSpec((TN, 128), lambda i: (i, 0)),
        out_shape=jax.ShapeDtypeStruct((N, 128), jnp.float32),
    )(hpart, x, norm, loop_weight, bias2)


def kernel(x, edge_index, edge_type, norm, weight, loop_weight, bias_parm):
    wr = weight.reshape(NUM_RELS * 128, 32)
    src = edge_index[0]
    dst = edge_index[1]
    pad = E_PAD - E
    src_r = jnp.concatenate([src, jnp.zeros((pad,), jnp.int32)]).reshape(NW, NCH, C)
    typ_r = jnp.concatenate([edge_type, jnp.zeros((pad,), jnp.int32)]).reshape(NW, NCH, C)
    dst_r = jnp.concatenate([dst, jnp.full((pad,), N, jnp.int32)]).reshape(NW, NCH, C)
    zblk = jnp.zeros((ZROWS, 128), jnp.float32)

    y = _make_y(x, wr)
    hpart = _make_sc(y, src_r, dst_r, typ_r, zblk)
    return _make_fin(hpart, x, norm, loop_weight, bias_parm.reshape(1, 128))


# speedup vs baseline: 25.5072x; 25.5072x over previous
import functools

import jax
import jax.numpy as jnp
from jax import lax
from jax.experimental import pallas as pl
from jax.experimental.pallas import tpu as pltpu
from jax.experimental.pallas import tpu_sc as plsc

N = 10000
E = 320000
IN_FEAT = 128
OUT_FEAT = 128
NUM_RELS = 90
NUM_BASES = 4
SUBMAT = 32

NC = 2
NS = 16
NW = NC * NS

C = 128
NCH = 80
PHASES = 4
PCH = NCH // PHASES
EPW = C * NCH
E_PAD = NW * EPW
N_ACC = N + 16
ZROWS = 1000

TN = 2000
NT = N // TN


def _y_body(w_ref, x_ref, y_ref):
    w = w_ref[...]
    wcat = jnp.concatenate([w, w, w, w], axis=1)
    ri = lax.broadcasted_iota(jnp.int32, (128, 128), 0)
    ci = lax.broadcasted_iota(jnp.int32, (128, 128), 1)
    mask = (ri // SUBMAT) == (ci // SUBMAT)
    wbd = jnp.where(mask, wcat, 0.0)
    y_ref[...] = jnp.dot(x_ref[...], wbd, preferred_element_type=jnp.float32)


def _make_y(x, wr):
    return pl.pallas_call(
        _y_body,
        grid=(NT, NUM_RELS),
        in_specs=[
            pl.BlockSpec((128, 32), lambda nt, r: (r, 0)),
            pl.BlockSpec((TN, 128), lambda nt, r: (nt, 0)),
        ],
        out_specs=pl.BlockSpec((TN, 128), lambda nt, r: (r * NT + nt, 0)),
        out_shape=jax.ShapeDtypeStruct((NUM_RELS * N, 128), jnp.float32),
    )(wr, x)


def _sc_body(y_hbm, src_hbm, dst_hbm, typ_hbm, z_hbm, out_hbm,
             src_v, typ_v, dst_v, idx_v, rows_a, rows_b, h_sh, sem_a, sem_b):
    cid = lax.axis_index("c")
    sid = lax.axis_index("s")
    wid = cid * NS + sid

    @pl.when(sid < 10)
    def _zero():
        pltpu.sync_copy(z_hbm, h_sh.at[pl.ds(sid * ZROWS, ZROWS)])
    plsc.subcore_barrier()

    def _gather(j, rows, sem):
        return pltpu.async_copy(y_hbm.at[idx_v.at[j]], rows, sem)

    def _wait(rows, sem):
        pltpu.make_async_copy(y_hbm.at[idx_v.at[0]], rows, sem).wait()

    def _scat(j, rows):
        pltpu.sync_copy(rows, h_sh.at[dst_v.at[j]], add=True)

    for p in range(PHASES):
        pltpu.sync_copy(src_hbm.at[wid].at[p], src_v)
        pltpu.sync_copy(typ_hbm.at[wid].at[p], typ_v)
        pltpu.sync_copy(dst_hbm.at[wid].at[p], dst_v)

        def _idx_row(j, carry):
            for k in range(C // 16):
                s = pl.ds(k * 16, 16)
                idx_v.at[j][s] = typ_v.at[j][s] * N + src_v.at[j][s]
            return carry
        lax.fori_loop(0, PCH, _idx_row, 0)

        _gather(0, rows_a, sem_a)
        def _loop(j2, carry):
            j0 = j2 * 2
            _gather(j0 + 1, rows_b, sem_b)
            _wait(rows_a, sem_a)
            _scat(j0, rows_a)
            @pl.when(j2 < PCH // 2 - 1)
            def _():
                _gather(j0 + 2, rows_a, sem_a)
            _wait(rows_b, sem_b)
            _scat(j0 + 1, rows_b)
            return carry
        lax.fori_loop(0, PCH // 2, _loop, 0)

    plsc.subcore_barrier()

    @pl.when(sid < 10)
    def _writeout():
        pltpu.sync_copy(h_sh.at[pl.ds(sid * ZROWS, ZROWS)],
                        out_hbm.at[cid].at[pl.ds(sid * ZROWS, ZROWS)])


def _make_sc(y, src_r, dst_r, typ_r, zblk):
    mesh = plsc.VectorSubcoreMesh(core_axis_name="c", subcore_axis_name="s")
    f = pl.kernel(
        _sc_body,
        out_type=jax.ShapeDtypeStruct((NC, N, 128), jnp.float32),
        mesh=mesh,
        scratch_types=[
            pltpu.VMEM((PCH, C), jnp.int32),
            pltpu.VMEM((PCH, C), jnp.int32),
            pltpu.VMEM((PCH, C), jnp.int32),
            pltpu.VMEM((PCH, C), jnp.int32),
            pltpu.VMEM((C, 128), jnp.float32),
            pltpu.VMEM((C, 128), jnp.float32),
            pltpu.VMEM_SHARED((N_ACC, 128), jnp.float32),
            pltpu.SemaphoreType.DMA,
            pltpu.SemaphoreType.DMA,
        ],
    )
    return f(y, src_r, dst_r, typ_r, zblk)


def _fin_body(h_ref, x_ref, norm_ref, lw_ref, b_ref, o_ref):
    h = h_ref[0] + h_ref[1]
    lm = jnp.dot(x_ref[...], lw_ref[...], preferred_element_type=jnp.float32)
    o_ref[...] = h * norm_ref[...] + b_ref[...] + lm


def _make_fin(hpart, x, norm, loop_weight, bias2):
    return pl.pallas_call(
        _fin_body,
        grid=(NT,),
        in_specs=[
            pl.BlockSpec((NC, TN, 128), lambda i: (0, i, 0)),
            pl.BlockSpec((TN, 128), lambda i: (i, 0)),
            pl.BlockSpec((TN, 1), lambda i: (i, 0)),
            pl.BlockSpec((128, 128), lambda i: (0, 0)),
            pl.BlockSpec((1, 128), lambda i: (0, 0)),
        ],
        out_specs=pl.BlockSpec((TN, 128), lambda i: (i, 0)),
        out_shape=jax.ShapeDtypeStruct((N, 128), jnp.float32),
    )(hpart, x, norm, loop_weight, bias2)


def kernel(x, edge_index, edge_type, norm, weight, loop_weight, bias_parm):
    wr = weight.reshape(NUM_RELS * 128, 32)
    src = edge_index[0]
    dst = edge_index[1]
    pad = E_PAD - E
    src_r = jnp.concatenate([src, jnp.zeros((pad,), jnp.int32)]).reshape(NW, PHASES, PCH, C)
    typ_r = jnp.concatenate([edge_type, jnp.zeros((pad,), jnp.int32)]).reshape(NW, PHASES, PCH, C)
    dst_r = jnp.concatenate([dst, jnp.full((pad,), N, jnp.int32)]).reshape(NW, PHASES, PCH, C)
    zblk = jnp.zeros((ZROWS, 128), jnp.float32)

    y = _make_y(x, wr)
    hpart = _make_sc(y, src_r, dst_r, typ_r, zblk)
    return _make_fin(hpart, x, norm, loop_weight, bias_parm.reshape(1, 128))
